# baseline (speedup 1.0000x reference)
import functools

import jax
import jax.numpy as jnp
from jax.experimental import pallas as pl
from jax.experimental.pallas import tpu as pltpu


def _vit_block_kernel(x_ref, ln1g_ref, ln1b_ref, qkvw_hbm, qkvb_ref,
                      projw_hbm, projb_ref, ln2g_ref, ln2b_ref,
                      fc1w_hbm, fc1b_ref, fc2w_hbm, fc2b_ref, o_ref,
                      wq_ref, wp_ref, w1_ref, w2_ref, stage_ref, sem,
                      *, n_heads, head_dim, n_img, t_pad, t_valid, eps):
    D = n_heads * head_dim
    hid = w1_ref.shape[1]
    M = n_img * t_pad

    @pl.when(pl.program_id(0) == 0)
    def _load_weights():
        def fetch(src, dst_slice):
            cp = pltpu.make_async_copy(src, dst_slice, sem)
            cp.start()
            cp.wait()
        for j in range(3):
            fetch(qkvw_hbm.at[:, j * D:(j + 1) * D], stage_ref.at[:, :])
            wq_ref[:, j * D:(j + 1) * D] = stage_ref[...].astype(jnp.bfloat16)
        fetch(projw_hbm, stage_ref.at[:, :])
        wp_ref[...] = stage_ref[...].astype(jnp.bfloat16)
        for j in range(hid // D):
            fetch(fc1w_hbm.at[:, j * D:(j + 1) * D], stage_ref.at[:, :])
            w1_ref[:, j * D:(j + 1) * D] = stage_ref[...].astype(jnp.bfloat16)
        for j in range(hid // D):
            fetch(fc2w_hbm.at[j * D:(j + 1) * D, :], stage_ref.at[:, :])
            w2_ref[j * D:(j + 1) * D, :] = stage_ref[...].astype(
                jnp.bfloat16)

    x = x_ref[...].reshape(M, D).astype(jnp.float32)

    mean = jnp.mean(x, axis=-1, keepdims=True)
    xc = x - mean
    var = jnp.mean(xc * xc, axis=-1, keepdims=True)
    xn = xc * jax.lax.rsqrt(var + eps) * ln1g_ref[...] + ln1b_ref[...]
    qkv = jnp.dot(xn.astype(jnp.bfloat16), wq_ref[...],
                  preferred_element_type=jnp.float32) + qkvb_ref[...]
    row = jax.lax.broadcasted_iota(jnp.int32, (M, 1), 0)
    valid = (row % t_pad) < t_valid
    q_bf = qkv[:, :D].astype(jnp.bfloat16)
    kv_bf = jnp.where(valid, qkv[:, D:], 0.0).astype(jnp.bfloat16)

    scale = jnp.float32(head_dim ** -0.5)
    attn_imgs = []
    for i in range(n_img):
        q_i = q_bf[i * t_pad:(i + 1) * t_pad]
        kv_i = kv_bf[i * t_pad:(i + 1) * t_pad]
        es = []
        denom = None
        for h in range(n_heads):
            qh = q_i[:, h * head_dim:(h + 1) * head_dim]
            kh = kv_i[:, h * head_dim:(h + 1) * head_dim]
            s = jax.lax.dot_general(qh, kh, (((1,), (1,)), ((), ())),
                                    preferred_element_type=jnp.float32)
            e = jnp.exp(s * scale)
            es.append(e)
            denom = e if denom is None else denom + e
        inv = pl.reciprocal(denom, approx=True)
        outs = []
        for h in range(n_heads):
            vh = kv_i[:, D + h * head_dim:D + (h + 1) * head_dim]
            attn_h = (es[h] * inv).astype(jnp.bfloat16)
            outs.append(jnp.dot(attn_h, vh,
                                preferred_element_type=jnp.float32))
        attn_imgs.append(jnp.concatenate(outs, axis=-1))
    attn = jnp.concatenate(attn_imgs, axis=0).astype(jnp.bfloat16)

    x2 = (jnp.dot(attn, wp_ref[...], preferred_element_type=jnp.float32)
          + projb_ref[...] + x)

    mean2 = jnp.mean(x2, axis=-1, keepdims=True)
    xc2 = x2 - mean2
    var2 = jnp.mean(xc2 * xc2, axis=-1, keepdims=True)
    xn2 = xc2 * jax.lax.rsqrt(var2 + eps) * ln2g_ref[...] + ln2b_ref[...]
    h1 = jnp.dot(xn2.astype(jnp.bfloat16), w1_ref[...],
                 preferred_element_type=jnp.float32) + fc1b_ref[...]
    c1 = jnp.float32(0.7978845608028654)
    c2 = jnp.float32(0.7978845608028654 * 0.044715)
    t = jnp.tanh(h1 * (c1 + c2 * (h1 * h1)))
    u = 0.5 * h1
    h1 = u + u * t
    out = (jnp.dot(h1.astype(jnp.bfloat16), w2_ref[...],
                   preferred_element_type=jnp.float32)
           + fc2b_ref[...] + x2)
    o_ref[...] = out.reshape(n_img, t_pad, D).astype(o_ref.dtype)


def kernel(x, ln1_g, ln1_b, qkv_w, qkv_b, proj_w, proj_b,
           ln2_g, ln2_b, fc1_w, fc1_b, fc2_w, fc2_b):
    B, T, D = x.shape
    n_heads = 12
    head_dim = D // n_heads
    hid = fc1_w.shape[1]
    Tp = ((T + 7) // 8) * 8
    n_img = 4 if B % 4 == 0 else (2 if B % 2 == 0 else 1)

    flops = B * (2 * Tp * D * 3 * D + 4 * n_heads * Tp * Tp * head_dim
                 + 2 * Tp * D * D + 4 * Tp * D * hid)
    transcend = B * (n_heads * Tp * Tp + Tp * hid)
    bytes_acc = (4 * B * T * D * 2
                 + 4 * (D * 3 * D + D * D + 2 * D * hid))

    out = pl.pallas_call(
        functools.partial(_vit_block_kernel, n_heads=n_heads,
                          head_dim=head_dim, n_img=n_img, t_pad=Tp,
                          t_valid=T, eps=1e-6),
        out_shape=jax.ShapeDtypeStruct((B, T, D), jnp.float32),
        grid=(B // n_img,),
        in_specs=[
            pl.BlockSpec((n_img, Tp, D), lambda b: (b, 0, 0)),
            pl.BlockSpec((1, D), lambda b: (0, 0)),
            pl.BlockSpec((1, D), lambda b: (0, 0)),
            pl.BlockSpec(memory_space=pl.ANY),
            pl.BlockSpec((1, 3 * D), lambda b: (0, 0)),
            pl.BlockSpec(memory_space=pl.ANY),
            pl.BlockSpec((1, D), lambda b: (0, 0)),
            pl.BlockSpec((1, D), lambda b: (0, 0)),
            pl.BlockSpec((1, D), lambda b: (0, 0)),
            pl.BlockSpec(memory_space=pl.ANY),
            pl.BlockSpec((1, hid), lambda b: (0, 0)),
            pl.BlockSpec(memory_space=pl.ANY),
            pl.BlockSpec((1, D), lambda b: (0, 0)),
        ],
        out_specs=pl.BlockSpec((n_img, Tp, D), lambda b: (b, 0, 0)),
        scratch_shapes=[
            pltpu.VMEM((D, 3 * D), jnp.bfloat16),
            pltpu.VMEM((D, D), jnp.bfloat16),
            pltpu.VMEM((D, hid), jnp.bfloat16),
            pltpu.VMEM((hid, D), jnp.bfloat16),
            pltpu.VMEM((D, D), jnp.float32),
            pltpu.SemaphoreType.DMA,
        ],
        compiler_params=pltpu.CompilerParams(
            dimension_semantics=("parallel",),
            vmem_limit_bytes=56 * 1024 * 1024,
        ),
        cost_estimate=pl.CostEstimate(flops=flops, transcendentals=transcend,
                                      bytes_accessed=bytes_acc),
    )(x, ln1_g.reshape(1, D), ln1_b.reshape(1, D),
      qkv_w, qkv_b.reshape(1, 3 * D),
      proj_w, proj_b.reshape(1, D),
      ln2_g.reshape(1, D), ln2_b.reshape(1, D),
      fc1_w, fc1_b.reshape(1, hid),
      fc2_w, fc2_b.reshape(1, D))
    return out


# speedup vs baseline: 1.0432x; 1.0432x over previous
import functools

import jax
import jax.numpy as jnp
from jax.experimental import pallas as pl
from jax.experimental.pallas import tpu as pltpu


def _vit_block_kernel(x_ref, ln1g_ref, ln1b_ref, qkvw_hbm, qkvb_ref,
                      projw_hbm, projb_ref, ln2g_ref, ln2b_ref,
                      fc1w_hbm, fc1b_ref, fc2w_hbm, fc2b_ref, o_ref,
                      wq_ref, wp_ref, w1_ref, w2_ref, stage_ref, sems,
                      *, n_heads, head_dim, n_img, t_pad, t_valid, eps):
    D = n_heads * head_dim
    hid = w1_ref.shape[1]
    M = n_img * t_pad

    @pl.when(pl.program_id(0) == 0)
    def _load_weights():
        cp_q = pltpu.make_async_copy(qkvw_hbm, stage_ref.at[:, :3 * D],
                                     sems.at[0])
        cp_p = pltpu.make_async_copy(projw_hbm, stage_ref.at[:, 3 * D:],
                                     sems.at[1])
        cp_q.start()
        cp_p.start()
        cp_q.wait()
        wq_ref[...] = stage_ref[:, :3 * D].astype(jnp.bfloat16)
        cp_p.wait()
        wp_ref[...] = stage_ref[:, 3 * D:].astype(jnp.bfloat16)
        cp_1 = pltpu.make_async_copy(fc1w_hbm, stage_ref.at[:, :],
                                     sems.at[0])
        cp_1.start()
        cp_1.wait()
        w1_ref[...] = stage_ref[...].astype(jnp.bfloat16)
        cps = [pltpu.make_async_copy(fc2w_hbm.at[j * D:(j + 1) * D, :],
                                     stage_ref.at[:, j * D:(j + 1) * D],
                                     sems.at[j])
               for j in range(hid // D)]
        for cp in cps:
            cp.start()
        for j, cp in enumerate(cps):
            cp.wait()
            w2_ref[j * D:(j + 1) * D, :] = (
                stage_ref[:, j * D:(j + 1) * D].astype(jnp.bfloat16))

    x = x_ref[...].reshape(M, D).astype(jnp.float32)

    mean = jnp.mean(x, axis=-1, keepdims=True)
    xc = x - mean
    var = jnp.mean(xc * xc, axis=-1, keepdims=True)
    xn = xc * jax.lax.rsqrt(var + eps) * ln1g_ref[...] + ln1b_ref[...]
    qkv = jnp.dot(xn.astype(jnp.bfloat16), wq_ref[...],
                  preferred_element_type=jnp.float32) + qkvb_ref[...]
    row = jax.lax.broadcasted_iota(jnp.int32, (M, 1), 0)
    valid = (row % t_pad) < t_valid
    q_bf = qkv[:, :D].astype(jnp.bfloat16)
    kv_bf = jnp.where(valid, qkv[:, D:], 0.0).astype(jnp.bfloat16)

    scale = jnp.float32(head_dim ** -0.5)
    attn_imgs = []
    for i in range(n_img):
        q_i = q_bf[i * t_pad:(i + 1) * t_pad]
        kv_i = kv_bf[i * t_pad:(i + 1) * t_pad]
        es = []
        denom = None
        for h in range(n_heads):
            qh = q_i[:, h * head_dim:(h + 1) * head_dim]
            kh = kv_i[:, h * head_dim:(h + 1) * head_dim]
            s = jax.lax.dot_general(qh, kh, (((1,), (1,)), ((), ())),
                                    preferred_element_type=jnp.float32)
            e = jnp.exp(s * scale)
            es.append(e)
            denom = e if denom is None else denom + e
        inv = pl.reciprocal(denom, approx=True)
        outs = []
        for h in range(n_heads):
            vh = kv_i[:, D + h * head_dim:D + (h + 1) * head_dim]
            attn_h = (es[h] * inv).astype(jnp.bfloat16)
            outs.append(jnp.dot(attn_h, vh,
                                preferred_element_type=jnp.float32))
        attn_imgs.append(jnp.concatenate(outs, axis=-1))
    attn = jnp.concatenate(attn_imgs, axis=0).astype(jnp.bfloat16)

    x2 = (jnp.dot(attn, wp_ref[...], preferred_element_type=jnp.float32)
          + projb_ref[...] + x)

    mean2 = jnp.mean(x2, axis=-1, keepdims=True)
    xc2 = x2 - mean2
    var2 = jnp.mean(xc2 * xc2, axis=-1, keepdims=True)
    xn2 = xc2 * jax.lax.rsqrt(var2 + eps) * ln2g_ref[...] + ln2b_ref[...]
    h1 = jnp.dot(xn2.astype(jnp.bfloat16), w1_ref[...],
                 preferred_element_type=jnp.float32) + fc1b_ref[...]
    c1 = jnp.float32(0.7978845608028654)
    c2 = jnp.float32(0.7978845608028654 * 0.044715)
    t = jnp.tanh(h1 * (c1 + c2 * (h1 * h1)))
    u = 0.5 * h1
    h1 = u + u * t
    out = (jnp.dot(h1.astype(jnp.bfloat16), w2_ref[...],
                   preferred_element_type=jnp.float32)
           + fc2b_ref[...] + x2)
    o_ref[...] = out.reshape(n_img, t_pad, D).astype(o_ref.dtype)


def kernel(x, ln1_g, ln1_b, qkv_w, qkv_b, proj_w, proj_b,
           ln2_g, ln2_b, fc1_w, fc1_b, fc2_w, fc2_b):
    B, T, D = x.shape
    n_heads = 12
    head_dim = D // n_heads
    hid = fc1_w.shape[1]
    Tp = ((T + 7) // 8) * 8
    n_img = 4 if B % 4 == 0 else (2 if B % 2 == 0 else 1)

    flops = B * (2 * Tp * D * 3 * D + 4 * n_heads * Tp * Tp * head_dim
                 + 2 * Tp * D * D + 4 * Tp * D * hid)
    transcend = B * (n_heads * Tp * Tp + Tp * hid)
    bytes_acc = (4 * B * T * D * 2
                 + 4 * (D * 3 * D + D * D + 2 * D * hid))

    out = pl.pallas_call(
        functools.partial(_vit_block_kernel, n_heads=n_heads,
                          head_dim=head_dim, n_img=n_img, t_pad=Tp,
                          t_valid=T, eps=1e-6),
        out_shape=jax.ShapeDtypeStruct((B, T, D), jnp.float32),
        grid=(B // n_img,),
        in_specs=[
            pl.BlockSpec((n_img, Tp, D), lambda b: (b, 0, 0)),
            pl.BlockSpec((1, D), lambda b: (0, 0)),
            pl.BlockSpec((1, D), lambda b: (0, 0)),
            pl.BlockSpec(memory_space=pl.ANY),
            pl.BlockSpec((1, 3 * D), lambda b: (0, 0)),
            pl.BlockSpec(memory_space=pl.ANY),
            pl.BlockSpec((1, D), lambda b: (0, 0)),
            pl.BlockSpec((1, D), lambda b: (0, 0)),
            pl.BlockSpec((1, D), lambda b: (0, 0)),
            pl.BlockSpec(memory_space=pl.ANY),
            pl.BlockSpec((1, hid), lambda b: (0, 0)),
            pl.BlockSpec(memory_space=pl.ANY),
            pl.BlockSpec((1, D), lambda b: (0, 0)),
        ],
        out_specs=pl.BlockSpec((n_img, Tp, D), lambda b: (b, 0, 0)),
        scratch_shapes=[
            pltpu.VMEM((D, 3 * D), jnp.bfloat16),
            pltpu.VMEM((D, D), jnp.bfloat16),
            pltpu.VMEM((D, hid), jnp.bfloat16),
            pltpu.VMEM((hid, D), jnp.bfloat16),
            pltpu.VMEM((D, hid), jnp.float32),
            pltpu.SemaphoreType.DMA((4,)),
        ],
        compiler_params=pltpu.CompilerParams(
            dimension_semantics=("parallel",),
            vmem_limit_bytes=56 * 1024 * 1024,
        ),
        cost_estimate=pl.CostEstimate(flops=flops, transcendentals=transcend,
                                      bytes_accessed=bytes_acc),
    )(x, ln1_g.reshape(1, D), ln1_b.reshape(1, D),
      qkv_w, qkv_b.reshape(1, 3 * D),
      proj_w, proj_b.reshape(1, D),
      ln2_g.reshape(1, D), ln2_b.reshape(1, D),
      fc1_w, fc1_b.reshape(1, hid),
      fc2_w, fc2_b.reshape(1, D))
    return out
